# baseline (speedup 1.0000x reference)
import functools

import jax
import jax.numpy as jnp
from jax import lax
from jax.experimental import pallas as pl
from jax.experimental.pallas import tpu as pltpu
from jax.experimental.pallas import tpu_sc as plsc

_NC = 2
_NS = 16
_NW = _NC * _NS

_H = 128
_NB = 2000
_EB = 6400
_CHG = 80
_CHS = 40
_NPAD = 128


def _node_pre_body(x_ref, pw_ref, pb_ref, h_ref):
    h = jnp.dot(x_ref[...], pw_ref[...], preferred_element_type=jnp.float32)
    h_ref[...] = h + pb_ref[...]


def _msg_body(gi_ref, gj_ref, ea_ref, ew_ref, eb_ref, w1c_ref, w1e_ref,
              b1_ref, w2_ref, b2_ref, out_ref):
    e = jnp.dot(ea_ref[...], ew_ref[...],
                preferred_element_type=jnp.float32) + eb_ref[...]
    gcat = jnp.concatenate([gi_ref[...], gj_ref[...]], axis=1)
    m = (jnp.dot(gcat, w1c_ref[...], preferred_element_type=jnp.float32)
         + jnp.dot(e, w1e_ref[...], preferred_element_type=jnp.float32)
         + b1_ref[...])
    m = jnp.maximum(m, 0.0)
    m = jnp.dot(m, w2_ref[...], preferred_element_type=jnp.float32) + b2_ref[...]
    out_ref[...] = jnp.maximum(m, 0.0)


def _upd_mid_body(h_ref, aa_ref, ab_ref, ac_ref, ad_ref,
                  u1_ref, bu1_ref, u2_ref, bu2_ref, hn_ref):
    aggr = aa_ref[...] + ab_ref[...] + ac_ref[...] + ad_ref[...]
    ucat = jnp.concatenate([h_ref[...], aggr], axis=1)
    u = jnp.dot(ucat, u1_ref[...], preferred_element_type=jnp.float32) + bu1_ref[...]
    u = jnp.maximum(u, 0.0)
    hn = jnp.dot(u, u2_ref[...], preferred_element_type=jnp.float32) + bu2_ref[...]
    hn_ref[...] = jnp.maximum(hn, 0.0)


def _upd_last_body(h_ref, aa_ref, ab_ref, ac_ref, ad_ref,
                   u1_ref, bu1_ref, u2_ref, bu2_ref,
                   t1_ref, tb1_ref, t2_ref, tb2_ref, hn_ref, lg_ref):
    aggr = aa_ref[...] + ab_ref[...] + ac_ref[...] + ad_ref[...]
    ucat = jnp.concatenate([h_ref[...], aggr], axis=1)
    u = jnp.dot(ucat, u1_ref[...], preferred_element_type=jnp.float32) + bu1_ref[...]
    u = jnp.maximum(u, 0.0)
    hn = jnp.dot(u, u2_ref[...], preferred_element_type=jnp.float32) + bu2_ref[...]
    hn = jnp.maximum(hn, 0.0)
    hn_ref[...] = hn
    t = jnp.dot(hn, t1_ref[...], preferred_element_type=jnp.float32) + tb1_ref[...]
    t = jnp.maximum(t, 0.0)
    lg_ref[...] = jnp.dot(t, t2_ref[...],
                          preferred_element_type=jnp.float32) + tb2_ref[...]


def _full(shape):
    return pl.BlockSpec(shape, lambda i: (0,) * len(shape))


def _rows(nrows, ncols):
    return pl.BlockSpec((nrows, ncols), lambda i: (i, 0))


_NBUF_G = 5
_NBUF_S = 5


def _sc_gather_body(h_hbm, dst_hbm, src_hbm, gi_hbm, gj_hbm,
                    di_v, si_v, ri_v, rj_v, sem_g, sem_w):
    e_total = gi_hbm.shape[0]
    epw = e_total // _NW
    nch = epw // _CHG
    wid = lax.axis_index("s") * _NC + lax.axis_index("c")
    pltpu.sync_copy(dst_hbm.at[pl.ds(wid * epw, epw)], di_v)
    pltpu.sync_copy(src_hbm.at[pl.ds(wid * epw, epw)], si_v)

    def start_gather(b, g):
        pltpu.async_copy(h_hbm.at[di_v.at[pl.ds(g * _CHG, _CHG)]], ri_v.at[b],
                         sem_g.at[b])
        pltpu.async_copy(h_hbm.at[si_v.at[pl.ds(g * _CHG, _CHG)]], rj_v.at[b],
                         sem_g.at[b])

    def wait_gather(b):
        pltpu.make_async_copy(h_hbm.at[di_v.at[pl.ds(0, _CHG)]], ri_v.at[b],
                              sem_g.at[b]).wait()
        pltpu.make_async_copy(h_hbm.at[si_v.at[pl.ds(0, _CHG)]], rj_v.at[b],
                              sem_g.at[b]).wait()

    for b in range(_NBUF_G):
        start_gather(b, b)

    def outer(i, carry):
        for b in range(_NBUF_G):
            g = i * _NBUF_G + b
            base = wid * epw + g * _CHG
            wait_gather(b)
            pltpu.async_copy(ri_v.at[b], gi_hbm.at[pl.ds(base, _CHG)],
                             sem_w.at[b])
            pltpu.async_copy(rj_v.at[b], gj_hbm.at[pl.ds(base, _CHG)],
                             sem_w.at[b])
        for b in range(_NBUF_G):
            g = i * _NBUF_G + b
            pltpu.make_async_copy(ri_v.at[b], gi_hbm.at[pl.ds(0, _CHG)],
                                  sem_w.at[b]).wait()
            pltpu.make_async_copy(rj_v.at[b], gj_hbm.at[pl.ds(0, _CHG)],
                                  sem_w.at[b]).wait()

            @pl.when(i < nch // _NBUF_G - 1)
            def _():
                start_gather(b, g + _NBUF_G)
        return carry

    lax.fori_loop(0, nch // _NBUF_G, outer, 0)


def _sc_scatter_body(m2_hbm, dst_hbm, zero_hbm, out_hbm, di_v, rows_v, acc_sp,
                     sem_l):
    e_total = m2_hbm.shape[0]
    n_total = zero_hbm.shape[0]
    cid = lax.axis_index("c")
    sid = lax.axis_index("s")
    npt = (n_total // _NS) // 8 * 8
    tail = n_total - npt * _NS
    eps = e_total // _NC
    ept = eps // _NS
    nch = ept // _CHS

    pltpu.sync_copy(zero_hbm.at[pl.ds(sid * npt, npt)],
                    acc_sp.at[pl.ds(sid * npt, npt)])
    if tail:
        @pl.when(sid == _NS - 1)
        def _():
            pltpu.sync_copy(zero_hbm.at[pl.ds(npt * _NS, tail)],
                            acc_sp.at[pl.ds(npt * _NS, tail)])
    plsc.subcore_barrier()

    def start_load(b, g):
        base = cid * eps + sid * ept + g * _CHS
        pltpu.async_copy(dst_hbm.at[pl.ds(base, _CHS)], di_v[b], sem_l.at[b])
        pltpu.async_copy(m2_hbm.at[pl.ds(base, _CHS)], rows_v.at[b],
                         sem_l.at[b])

    for b in range(_NBUF_S):
        start_load(b, b)

    def outer(i, carry):
        for b in range(_NBUF_S):
            g = i * _NBUF_S + b
            pltpu.make_async_copy(dst_hbm.at[pl.ds(0, _CHS)], di_v[b],
                                  sem_l.at[b]).wait()
            pltpu.make_async_copy(m2_hbm.at[pl.ds(0, _CHS)], rows_v.at[b],
                                  sem_l.at[b]).wait()
            pltpu.sync_copy(rows_v.at[b], acc_sp.at[di_v[b]], add=True)

            @pl.when(i < nch // _NBUF_S - 1)
            def _():
                start_load(b, g + _NBUF_S)
        return carry

    lax.fori_loop(0, nch // _NBUF_S, outer, 0)
    plsc.subcore_barrier()
    pltpu.sync_copy(acc_sp.at[pl.ds(sid * npt, npt)],
                    out_hbm.at[cid].at[pl.ds(sid * npt, npt)])
    if tail:
        @pl.when(sid == _NS - 1)
        def _():
            pltpu.sync_copy(acc_sp.at[pl.ds(npt * _NS, tail)],
                            out_hbm.at[cid].at[pl.ds(npt * _NS, tail)])


def _run(x, src, dst, edge_attr, params):
    n, d = x.shape
    e = src.shape[0]
    de = edge_attr.shape[1]
    f32 = jnp.float32

    pw, pb = params['node_proj']
    ew, eb = params['edge_proj']
    layers = params['layers']
    tw1, tb1 = params['tok1']
    tw2, tb2 = params['tok2']

    def r2(v):
        return v.reshape(1, -1)

    nb = n // _NB

    node_pre = pl.pallas_call(
        _node_pre_body,
        grid=(nb,),
        in_specs=[_rows(_NB, d), _full((d, _H)), _full((1, _H))],
        out_specs=_rows(_NB, _H),
        out_shape=jax.ShapeDtypeStruct((n, _H), f32),
    )
    h = node_pre(x, pw, r2(pb))

    mesh = plsc.VectorSubcoreMesh(core_axis_name="c", subcore_axis_name="s")

    eh = e // 2
    unit = _NW * _CHG * _NBUF_G
    ehp = -(-eh // unit) * unit

    sc_gather = functools.partial(
        pl.kernel,
        out_type=[jax.ShapeDtypeStruct((ehp, _H), f32)] * 2,
        mesh=mesh,
        scratch_types=[
            pltpu.VMEM((ehp // _NW,), jnp.int32),
            pltpu.VMEM((ehp // _NW,), jnp.int32),
            pltpu.VMEM((_NBUF_G, _CHG, _H), f32),
            pltpu.VMEM((_NBUF_G, _CHG, _H), f32),
            pltpu.SemaphoreType.DMA((_NBUF_G,)),
            pltpu.SemaphoreType.DMA((_NBUF_G,)),
        ],
    )(_sc_gather_body)

    sc_scatter = functools.partial(
        pl.kernel,
        out_type=jax.ShapeDtypeStruct((_NC, n, _H), f32),
        mesh=mesh,
        scratch_types=[
            [pltpu.VMEM((_CHS,), jnp.int32)] * _NBUF_S,
            pltpu.VMEM((_NBUF_S, _CHS, _H), f32),
            pltpu.VMEM_SHARED((n + _NPAD, _H), f32),
            pltpu.SemaphoreType.DMA((_NBUF_S,)),
        ],
    )(_sc_scatter_body)

    msg = pl.pallas_call(
        _msg_body,
        grid=(ehp // _EB,),
        in_specs=[_rows(_EB, _H), _rows(_EB, _H), _rows(_EB, de),
                  _full((de, de)), _full((1, de)),
                  _full((2 * _H, _H)), _full((de, _H)), _full((1, _H)),
                  _full((_H, _H)), _full((1, _H))],
        out_specs=_rows(_EB, _H),
        out_shape=jax.ShapeDtypeStruct((ehp, _H), f32),
    )

    upd_mid = pl.pallas_call(
        _upd_mid_body,
        grid=(nb,),
        in_specs=[_rows(_NB, _H)] * 5 + [
            _full((2 * _H, _H)), _full((1, _H)),
            _full((_H, _H)), _full((1, _H))],
        out_specs=_rows(_NB, _H),
        out_shape=jax.ShapeDtypeStruct((n, _H), f32),
    )

    hh = _H // 2
    upd_last = pl.pallas_call(
        _upd_last_body,
        grid=(nb,),
        in_specs=[_rows(_NB, _H)] * 5 + [
            _full((2 * _H, _H)), _full((1, _H)),
            _full((_H, _H)), _full((1, _H)),
            _full((_H, hh)), _full((1, hh)), _full((hh, 1)), _full((1, 1))],
        out_specs=[_rows(_NB, _H), _rows(_NB, 1)],
        out_shape=[jax.ShapeDtypeStruct((n, _H), f32),
                   jax.ShapeDtypeStruct((n, 1), f32)],
    )

    zero_n = jnp.zeros((n, _H), f32)
    npad = ehp - eh
    padi = jnp.zeros((npad,), jnp.int32)
    pads = n + jax.lax.rem(jax.lax.iota(jnp.int32, npad), _NPAD)
    pade = jnp.zeros((npad, de), f32)
    dstg = (jnp.concatenate([dst[:eh], padi]), jnp.concatenate([dst[eh:], padi]))
    dsts = (jnp.concatenate([dst[:eh], pads]), jnp.concatenate([dst[eh:], pads]))
    srcg = (jnp.concatenate([src[:eh], padi]), jnp.concatenate([src[eh:], padi]))
    eah = (jnp.concatenate([edge_attr[:eh], pade]),
           jnp.concatenate([edge_attr[eh:], pade]))

    for li, layer in enumerate(layers):
        w1, b1 = layer['msg1']
        w2, b2 = layer['msg2']
        g0 = sc_gather(h, dstg[0], srcg[0])
        g1 = sc_gather(h, dstg[1], srcg[1])
        m2_0 = msg(g0[0], g0[1], eah[0], ew, r2(eb), w1[:2 * _H],
                   w1[2 * _H:], r2(b1), w2, r2(b2))
        m2_1 = msg(g1[0], g1[1], eah[1], ew, r2(eb), w1[:2 * _H],
                   w1[2 * _H:], r2(b1), w2, r2(b2))
        ag0 = sc_scatter(m2_0, dsts[0], zero_n)
        ag1 = sc_scatter(m2_1, dsts[1], zero_n)
        uw1, ub1 = layer['up1']
        uw2, ub2 = layer['up2']
        if li < 2:
            h = upd_mid(h, ag0[0], ag0[1], ag1[0], ag1[1],
                        uw1, r2(ub1), uw2, r2(ub2))
        else:
            h, lg = upd_last(h, ag0[0], ag0[1], ag1[0], ag1[1],
                             uw1, r2(ub1), uw2, r2(ub2),
                             tw1, r2(tb1), tw2, r2(tb2))

    return lg.reshape(-1), h


def kernel(x, edge_index, edge_attr, params):
    src = edge_index[0]
    dst = edge_index[1]
    return _run(x, src, dst, edge_attr, params)


# speedup vs baseline: 3.0877x; 3.0877x over previous
import functools

import jax
import jax.numpy as jnp
from jax import lax
from jax.experimental import pallas as pl
from jax.experimental.pallas import tpu as pltpu
from jax.experimental.pallas import tpu_sc as plsc

_NC = 2
_NS = 16
_NW = _NC * _NS

_H = 128
_NB = 2000
_EB = 8000
_CHG = 40
_CHS = 40
_NPAD = 128


def _node_pre_body(x_ref, pw_ref, pb_ref, h_ref):
    h = jnp.dot(x_ref[...], pw_ref[...], preferred_element_type=jnp.float32)
    h_ref[...] = h + pb_ref[...]


def _msg_body(gi_ref, gj_ref, ea_ref, ew_ref, eb_ref, w1c_ref, w1e_ref,
              b1_ref, w2_ref, b2_ref, out_ref):
    e = jnp.dot(ea_ref[...], ew_ref[...],
                preferred_element_type=jnp.float32) + eb_ref[...]
    gcat = jnp.concatenate([gi_ref[...], gj_ref[...]], axis=1)
    m = (jnp.dot(gcat, w1c_ref[...], preferred_element_type=jnp.float32)
         + jnp.dot(e, w1e_ref[...], preferred_element_type=jnp.float32)
         + b1_ref[...])
    m = jnp.maximum(m, 0.0)
    m = jnp.dot(m, w2_ref[...], preferred_element_type=jnp.float32) + b2_ref[...]
    out_ref[...] = jnp.maximum(m, 0.0)


def _upd_mid_body(h_ref, aa_ref, ab_ref, ac_ref, ad_ref,
                  u1_ref, bu1_ref, u2_ref, bu2_ref, hn_ref):
    aggr = aa_ref[...] + ab_ref[...] + ac_ref[...] + ad_ref[...]
    ucat = jnp.concatenate([h_ref[...], aggr], axis=1)
    u = jnp.dot(ucat, u1_ref[...], preferred_element_type=jnp.float32) + bu1_ref[...]
    u = jnp.maximum(u, 0.0)
    hn = jnp.dot(u, u2_ref[...], preferred_element_type=jnp.float32) + bu2_ref[...]
    hn_ref[...] = jnp.maximum(hn, 0.0)


def _upd_last_body(h_ref, aa_ref, ab_ref, ac_ref, ad_ref,
                   u1_ref, bu1_ref, u2_ref, bu2_ref,
                   t1_ref, tb1_ref, t2_ref, tb2_ref, hn_ref, lg_ref):
    aggr = aa_ref[...] + ab_ref[...] + ac_ref[...] + ad_ref[...]
    ucat = jnp.concatenate([h_ref[...], aggr], axis=1)
    u = jnp.dot(ucat, u1_ref[...], preferred_element_type=jnp.float32) + bu1_ref[...]
    u = jnp.maximum(u, 0.0)
    hn = jnp.dot(u, u2_ref[...], preferred_element_type=jnp.float32) + bu2_ref[...]
    hn = jnp.maximum(hn, 0.0)
    hn_ref[...] = hn
    t = jnp.dot(hn, t1_ref[...], preferred_element_type=jnp.float32) + tb1_ref[...]
    t = jnp.maximum(t, 0.0)
    lg_ref[...] = jnp.dot(t, t2_ref[...],
                          preferred_element_type=jnp.float32) + tb2_ref[...]


def _full(shape):
    return pl.BlockSpec(shape, lambda i: (0,) * len(shape))


def _rows(nrows, ncols):
    return pl.BlockSpec((nrows, ncols), lambda i: (i, 0))


_NBUF_G = 5
_NBUF_S = 5


def _sc_gather_body(h_hbm, dst_hbm, src_hbm, gi_hbm, gj_hbm,
                    di_v, si_v, ri_v, rj_v, sem_g, sem_w):
    e_total = gi_hbm.shape[0]
    epw = e_total // _NW
    nch = epw // _CHG
    wid = lax.axis_index("s") * _NC + lax.axis_index("c")
    pltpu.sync_copy(dst_hbm.at[pl.ds(wid * epw, epw)], di_v)
    pltpu.sync_copy(src_hbm.at[pl.ds(wid * epw, epw)], si_v)

    def start_gather(b, g):
        pltpu.async_copy(h_hbm.at[di_v.at[pl.ds(g * _CHG, _CHG)]], ri_v.at[b],
                         sem_g.at[b])
        pltpu.async_copy(h_hbm.at[si_v.at[pl.ds(g * _CHG, _CHG)]], rj_v.at[b],
                         sem_g.at[b])

    def wait_gather(b):
        pltpu.make_async_copy(h_hbm.at[di_v.at[pl.ds(0, _CHG)]], ri_v.at[b],
                              sem_g.at[b]).wait()
        pltpu.make_async_copy(h_hbm.at[si_v.at[pl.ds(0, _CHG)]], rj_v.at[b],
                              sem_g.at[b]).wait()

    for b in range(_NBUF_G):
        start_gather(b, b)

    def outer(i, carry):
        for b in range(_NBUF_G):
            g = i * _NBUF_G + b
            base = wid * epw + g * _CHG
            wait_gather(b)
            pltpu.async_copy(ri_v.at[b], gi_hbm.at[pl.ds(base, _CHG)],
                             sem_w.at[b])
            pltpu.async_copy(rj_v.at[b], gj_hbm.at[pl.ds(base, _CHG)],
                             sem_w.at[b])
        for b in range(_NBUF_G):
            g = i * _NBUF_G + b
            pltpu.make_async_copy(ri_v.at[b], gi_hbm.at[pl.ds(0, _CHG)],
                                  sem_w.at[b]).wait()
            pltpu.make_async_copy(rj_v.at[b], gj_hbm.at[pl.ds(0, _CHG)],
                                  sem_w.at[b]).wait()

            @pl.when(i < nch // _NBUF_G - 1)
            def _():
                start_gather(b, g + _NBUF_G)
        return carry

    lax.fori_loop(0, nch // _NBUF_G, outer, 0)


def _sc_scatter_body(m2_hbm, dst_hbm, zero_hbm, out_hbm, di_v, rows_v, acc_sp,
                     sem_l):
    e_total = m2_hbm.shape[0]
    n_total = zero_hbm.shape[0]
    cid = lax.axis_index("c")
    sid = lax.axis_index("s")
    npt = (n_total // _NS) // 8 * 8
    tail = n_total - npt * _NS
    eps = e_total // _NC
    ept = eps // _NS
    nch = ept // _CHS

    pltpu.sync_copy(zero_hbm.at[pl.ds(sid * npt, npt)],
                    acc_sp.at[pl.ds(sid * npt, npt)])
    if tail:
        @pl.when(sid == _NS - 1)
        def _():
            pltpu.sync_copy(zero_hbm.at[pl.ds(npt * _NS, tail)],
                            acc_sp.at[pl.ds(npt * _NS, tail)])
    plsc.subcore_barrier()

    def start_load(b, g):
        base = cid * eps + sid * ept + g * _CHS
        pltpu.async_copy(dst_hbm.at[pl.ds(base, _CHS)], di_v[b], sem_l.at[b])
        pltpu.async_copy(m2_hbm.at[pl.ds(base, _CHS)], rows_v.at[b],
                         sem_l.at[b])

    for b in range(_NBUF_S):
        start_load(b, b)

    def outer(i, carry):
        for b in range(_NBUF_S):
            g = i * _NBUF_S + b
            pltpu.make_async_copy(dst_hbm.at[pl.ds(0, _CHS)], di_v[b],
                                  sem_l.at[b]).wait()
            pltpu.make_async_copy(m2_hbm.at[pl.ds(0, _CHS)], rows_v.at[b],
                                  sem_l.at[b]).wait()
            pltpu.sync_copy(rows_v.at[b], acc_sp.at[di_v[b]], add=True)

            @pl.when(i < nch // _NBUF_S - 1)
            def _():
                start_load(b, g + _NBUF_S)
        return carry

    lax.fori_loop(0, nch // _NBUF_S, outer, 0)
    plsc.subcore_barrier()
    pltpu.sync_copy(acc_sp.at[pl.ds(sid * npt, npt)],
                    out_hbm.at[cid].at[pl.ds(sid * npt, npt)])
    if tail:
        @pl.when(sid == _NS - 1)
        def _():
            pltpu.sync_copy(acc_sp.at[pl.ds(npt * _NS, tail)],
                            out_hbm.at[cid].at[pl.ds(npt * _NS, tail)])


def _run(x, src, dst, edge_attr, params):
    n, d = x.shape
    e = src.shape[0]
    de = edge_attr.shape[1]
    f32 = jnp.float32

    pw, pb = params['node_proj']
    ew, eb = params['edge_proj']
    layers = params['layers']
    tw1, tb1 = params['tok1']
    tw2, tb2 = params['tok2']

    def r2(v):
        return v.reshape(1, -1)

    nb = n // _NB

    node_pre = pl.pallas_call(
        _node_pre_body,
        grid=(nb,),
        in_specs=[_rows(_NB, d), _full((d, _H)), _full((1, _H))],
        out_specs=_rows(_NB, _H),
        out_shape=jax.ShapeDtypeStruct((n, _H), f32),
    )
    h = node_pre(x, pw, r2(pb))

    mesh = plsc.VectorSubcoreMesh(core_axis_name="c", subcore_axis_name="s")

    eh = e // 2
    unit = _NW * _CHG * _NBUF_G
    ehp = -(-eh // unit) * unit

    sc_gather = functools.partial(
        pl.kernel,
        out_type=[jax.ShapeDtypeStruct((ehp, _H), f32)] * 2,
        mesh=mesh,
        scratch_types=[
            pltpu.VMEM((ehp // _NW,), jnp.int32),
            pltpu.VMEM((ehp // _NW,), jnp.int32),
            pltpu.VMEM((_NBUF_G, _CHG, _H), f32),
            pltpu.VMEM((_NBUF_G, _CHG, _H), f32),
            pltpu.SemaphoreType.DMA((_NBUF_G,)),
            pltpu.SemaphoreType.DMA((_NBUF_G,)),
        ],
    )(_sc_gather_body)

    sc_scatter = functools.partial(
        pl.kernel,
        out_type=jax.ShapeDtypeStruct((_NC, n, _H), f32),
        mesh=mesh,
        scratch_types=[
            [pltpu.VMEM((_CHS,), jnp.int32)] * _NBUF_S,
            pltpu.VMEM((_NBUF_S, _CHS, _H), f32),
            pltpu.VMEM_SHARED((n + _NPAD, _H), f32),
            pltpu.SemaphoreType.DMA((_NBUF_S,)),
        ],
    )(_sc_scatter_body)

    msg = pl.pallas_call(
        _msg_body,
        grid=(ehp // _EB,),
        in_specs=[_rows(_EB, _H), _rows(_EB, _H), _rows(_EB, de),
                  _full((de, de)), _full((1, de)),
                  _full((2 * _H, _H)), _full((de, _H)), _full((1, _H)),
                  _full((_H, _H)), _full((1, _H))],
        out_specs=_rows(_EB, _H),
        out_shape=jax.ShapeDtypeStruct((ehp, _H), f32),
    )

    upd_mid = pl.pallas_call(
        _upd_mid_body,
        grid=(nb,),
        in_specs=[_rows(_NB, _H)] * 5 + [
            _full((2 * _H, _H)), _full((1, _H)),
            _full((_H, _H)), _full((1, _H))],
        out_specs=_rows(_NB, _H),
        out_shape=jax.ShapeDtypeStruct((n, _H), f32),
    )

    hh = _H // 2
    upd_last = pl.pallas_call(
        _upd_last_body,
        grid=(nb,),
        in_specs=[_rows(_NB, _H)] * 5 + [
            _full((2 * _H, _H)), _full((1, _H)),
            _full((_H, _H)), _full((1, _H)),
            _full((_H, hh)), _full((1, hh)), _full((hh, 1)), _full((1, 1))],
        out_specs=[_rows(_NB, _H), _rows(_NB, 1)],
        out_shape=[jax.ShapeDtypeStruct((n, _H), f32),
                   jax.ShapeDtypeStruct((n, 1), f32)],
    )

    zero_n = jnp.zeros((n, _H), f32)
    npad = ehp - eh
    padi = jnp.zeros((npad,), jnp.int32)
    pads = n + jax.lax.rem(jax.lax.iota(jnp.int32, npad), _NPAD)
    pade = jnp.zeros((npad, de), f32)
    dstg = (jnp.concatenate([dst[:eh], padi]), jnp.concatenate([dst[eh:], padi]))
    dsts = (jnp.concatenate([dst[:eh], pads]), jnp.concatenate([dst[eh:], pads]))
    srcg = (jnp.concatenate([src[:eh], padi]), jnp.concatenate([src[eh:], padi]))
    eah = (jnp.concatenate([edge_attr[:eh], pade]),
           jnp.concatenate([edge_attr[eh:], pade]))

    for li, layer in enumerate(layers):
        w1, b1 = layer['msg1']
        w2, b2 = layer['msg2']
        g0 = sc_gather(h, dstg[0], srcg[0])
        g1 = sc_gather(h, dstg[1], srcg[1])
        m2_0 = msg(g0[0], g0[1], eah[0], ew, r2(eb), w1[:2 * _H],
                   w1[2 * _H:], r2(b1), w2, r2(b2))
        m2_1 = msg(g1[0], g1[1], eah[1], ew, r2(eb), w1[:2 * _H],
                   w1[2 * _H:], r2(b1), w2, r2(b2))
        ag0 = sc_scatter(m2_0, dsts[0], zero_n)
        ag1 = sc_scatter(m2_1, dsts[1], zero_n)
        uw1, ub1 = layer['up1']
        uw2, ub2 = layer['up2']
        if li < 2:
            h = upd_mid(h, ag0[0], ag0[1], ag1[0], ag1[1],
                        uw1, r2(ub1), uw2, r2(ub2))
        else:
            h, lg = upd_last(h, ag0[0], ag0[1], ag1[0], ag1[1],
                             uw1, r2(ub1), uw2, r2(ub2),
                             tw1, r2(tb1), tw2, r2(tb2))

    return lg.reshape(-1), h


def kernel(x, edge_index, edge_attr, params):
    src = edge_index[0]
    dst = edge_index[1]
    return _run(x, src, dst, edge_attr, params)


# speedup vs baseline: 3.0923x; 1.0015x over previous
import functools

import jax
import jax.numpy as jnp
from jax import lax
from jax.experimental import pallas as pl
from jax.experimental.pallas import tpu as pltpu
from jax.experimental.pallas import tpu_sc as plsc

_NC = 2
_NS = 16
_NW = _NC * _NS

_H = 128
_NB = 2000
_EB = 6400
_CHG = 40
_CHS = 40
_NPAD = 128


def _node_pre_body(x_ref, pw_ref, pb_ref, h_ref):
    h = jnp.dot(x_ref[...], pw_ref[...], preferred_element_type=jnp.float32)
    h_ref[...] = h + pb_ref[...]


def _msg_body(gi_ref, gj_ref, ea_ref, ew_ref, eb_ref, w1c_ref, w1e_ref,
              b1_ref, w2_ref, b2_ref, out_ref):
    e = jnp.dot(ea_ref[...], ew_ref[...],
                preferred_element_type=jnp.float32) + eb_ref[...]
    gcat = jnp.concatenate([gi_ref[...], gj_ref[...]], axis=1)
    m = (jnp.dot(gcat, w1c_ref[...], preferred_element_type=jnp.float32)
         + jnp.dot(e, w1e_ref[...], preferred_element_type=jnp.float32)
         + b1_ref[...])
    m = jnp.maximum(m, 0.0)
    m = jnp.dot(m, w2_ref[...], preferred_element_type=jnp.float32) + b2_ref[...]
    out_ref[...] = jnp.maximum(m, 0.0)


def _upd_mid_body(h_ref, aa_ref, ab_ref, ac_ref, ad_ref,
                  u1_ref, bu1_ref, u2_ref, bu2_ref, hn_ref):
    aggr = aa_ref[...] + ab_ref[...] + ac_ref[...] + ad_ref[...]
    ucat = jnp.concatenate([h_ref[...], aggr], axis=1)
    u = jnp.dot(ucat, u1_ref[...], preferred_element_type=jnp.float32) + bu1_ref[...]
    u = jnp.maximum(u, 0.0)
    hn = jnp.dot(u, u2_ref[...], preferred_element_type=jnp.float32) + bu2_ref[...]
    hn_ref[...] = jnp.maximum(hn, 0.0)


def _upd_last_body(h_ref, aa_ref, ab_ref, ac_ref, ad_ref,
                   u1_ref, bu1_ref, u2_ref, bu2_ref,
                   t1_ref, tb1_ref, t2_ref, tb2_ref, hn_ref, lg_ref):
    aggr = aa_ref[...] + ab_ref[...] + ac_ref[...] + ad_ref[...]
    ucat = jnp.concatenate([h_ref[...], aggr], axis=1)
    u = jnp.dot(ucat, u1_ref[...], preferred_element_type=jnp.float32) + bu1_ref[...]
    u = jnp.maximum(u, 0.0)
    hn = jnp.dot(u, u2_ref[...], preferred_element_type=jnp.float32) + bu2_ref[...]
    hn = jnp.maximum(hn, 0.0)
    hn_ref[...] = hn
    t = jnp.dot(hn, t1_ref[...], preferred_element_type=jnp.float32) + tb1_ref[...]
    t = jnp.maximum(t, 0.0)
    lg_ref[...] = jnp.dot(t, t2_ref[...],
                          preferred_element_type=jnp.float32) + tb2_ref[...]


def _full(shape):
    return pl.BlockSpec(shape, lambda i: (0,) * len(shape))


def _rows(nrows, ncols):
    return pl.BlockSpec((nrows, ncols), lambda i: (i, 0))


_NBUF_G = 5
_NBUF_S = 5


def _sc_gather_body(h_hbm, dst_hbm, src_hbm, gi_hbm, gj_hbm,
                    di_v, si_v, ri_v, rj_v, sem_g, sem_w):
    e_total = gi_hbm.shape[0]
    epw = e_total // _NW
    nch = epw // _CHG
    wid = lax.axis_index("s") * _NC + lax.axis_index("c")
    pltpu.sync_copy(dst_hbm.at[pl.ds(wid * epw, epw)], di_v)
    pltpu.sync_copy(src_hbm.at[pl.ds(wid * epw, epw)], si_v)

    def start_gather(b, g):
        pltpu.async_copy(h_hbm.at[di_v.at[pl.ds(g * _CHG, _CHG)]], ri_v.at[b],
                         sem_g.at[b])
        pltpu.async_copy(h_hbm.at[si_v.at[pl.ds(g * _CHG, _CHG)]], rj_v.at[b],
                         sem_g.at[b])

    def wait_gather(b):
        pltpu.make_async_copy(h_hbm.at[di_v.at[pl.ds(0, _CHG)]], ri_v.at[b],
                              sem_g.at[b]).wait()
        pltpu.make_async_copy(h_hbm.at[si_v.at[pl.ds(0, _CHG)]], rj_v.at[b],
                              sem_g.at[b]).wait()

    for b in range(_NBUF_G):
        start_gather(b, b)

    def outer(i, carry):
        for b in range(_NBUF_G):
            g = i * _NBUF_G + b
            base = wid * epw + g * _CHG
            wait_gather(b)
            pltpu.async_copy(ri_v.at[b], gi_hbm.at[pl.ds(base, _CHG)],
                             sem_w.at[b])
            pltpu.async_copy(rj_v.at[b], gj_hbm.at[pl.ds(base, _CHG)],
                             sem_w.at[b])
        for b in range(_NBUF_G):
            g = i * _NBUF_G + b
            pltpu.make_async_copy(ri_v.at[b], gi_hbm.at[pl.ds(0, _CHG)],
                                  sem_w.at[b]).wait()
            pltpu.make_async_copy(rj_v.at[b], gj_hbm.at[pl.ds(0, _CHG)],
                                  sem_w.at[b]).wait()

            @pl.when(i < nch // _NBUF_G - 1)
            def _():
                start_gather(b, g + _NBUF_G)
        return carry

    lax.fori_loop(0, nch // _NBUF_G, outer, 0)


def _sc_scatter_body(m2_hbm, dst_hbm, zero_hbm, out_hbm, di_v, rows_v, acc_sp,
                     sem_l):
    e_total = m2_hbm.shape[0]
    n_total = zero_hbm.shape[0]
    cid = lax.axis_index("c")
    sid = lax.axis_index("s")
    npt = (n_total // _NS) // 8 * 8
    tail = n_total - npt * _NS
    eps = e_total // _NC
    ept = eps // _NS
    nch = ept // _CHS

    pltpu.sync_copy(zero_hbm.at[pl.ds(sid * npt, npt)],
                    acc_sp.at[pl.ds(sid * npt, npt)])
    if tail:
        @pl.when(sid == _NS - 1)
        def _():
            pltpu.sync_copy(zero_hbm.at[pl.ds(npt * _NS, tail)],
                            acc_sp.at[pl.ds(npt * _NS, tail)])
    plsc.subcore_barrier()

    def start_load(b, g):
        base = cid * eps + sid * ept + g * _CHS
        pltpu.async_copy(dst_hbm.at[pl.ds(base, _CHS)], di_v[b], sem_l.at[b])
        pltpu.async_copy(m2_hbm.at[pl.ds(base, _CHS)], rows_v.at[b],
                         sem_l.at[b])

    for b in range(_NBUF_S):
        start_load(b, b)

    def outer(i, carry):
        for b in range(_NBUF_S):
            g = i * _NBUF_S + b
            pltpu.make_async_copy(dst_hbm.at[pl.ds(0, _CHS)], di_v[b],
                                  sem_l.at[b]).wait()
            pltpu.make_async_copy(m2_hbm.at[pl.ds(0, _CHS)], rows_v.at[b],
                                  sem_l.at[b]).wait()
            pltpu.sync_copy(rows_v.at[b], acc_sp.at[di_v[b]], add=True)

            @pl.when(i < nch // _NBUF_S - 1)
            def _():
                start_load(b, g + _NBUF_S)
        return carry

    lax.fori_loop(0, nch // _NBUF_S, outer, 0)
    plsc.subcore_barrier()
    pltpu.sync_copy(acc_sp.at[pl.ds(sid * npt, npt)],
                    out_hbm.at[cid].at[pl.ds(sid * npt, npt)])
    if tail:
        @pl.when(sid == _NS - 1)
        def _():
            pltpu.sync_copy(acc_sp.at[pl.ds(npt * _NS, tail)],
                            out_hbm.at[cid].at[pl.ds(npt * _NS, tail)])


def _run(x, src, dst, edge_attr, params):
    n, d = x.shape
    e = src.shape[0]
    de = edge_attr.shape[1]
    f32 = jnp.float32

    pw, pb = params['node_proj']
    ew, eb = params['edge_proj']
    layers = params['layers']
    tw1, tb1 = params['tok1']
    tw2, tb2 = params['tok2']

    def r2(v):
        return v.reshape(1, -1)

    nb = n // _NB

    node_pre = pl.pallas_call(
        _node_pre_body,
        grid=(nb,),
        in_specs=[_rows(_NB, d), _full((d, _H)), _full((1, _H))],
        out_specs=_rows(_NB, _H),
        out_shape=jax.ShapeDtypeStruct((n, _H), f32),
    )
    h = node_pre(x, pw, r2(pb))

    mesh = plsc.VectorSubcoreMesh(core_axis_name="c", subcore_axis_name="s")

    eh = e // 2
    unit = _NW * _CHG * _NBUF_G
    ehp = -(-eh // unit) * unit

    sc_gather = functools.partial(
        pl.kernel,
        out_type=[jax.ShapeDtypeStruct((ehp, _H), f32)] * 2,
        mesh=mesh,
        scratch_types=[
            pltpu.VMEM((ehp // _NW,), jnp.int32),
            pltpu.VMEM((ehp // _NW,), jnp.int32),
            pltpu.VMEM((_NBUF_G, _CHG, _H), f32),
            pltpu.VMEM((_NBUF_G, _CHG, _H), f32),
            pltpu.SemaphoreType.DMA((_NBUF_G,)),
            pltpu.SemaphoreType.DMA((_NBUF_G,)),
        ],
    )(_sc_gather_body)

    sc_scatter = functools.partial(
        pl.kernel,
        out_type=jax.ShapeDtypeStruct((_NC, n, _H), f32),
        mesh=mesh,
        scratch_types=[
            [pltpu.VMEM((_CHS,), jnp.int32)] * _NBUF_S,
            pltpu.VMEM((_NBUF_S, _CHS, _H), f32),
            pltpu.VMEM_SHARED((n + _NPAD, _H), f32),
            pltpu.SemaphoreType.DMA((_NBUF_S,)),
        ],
    )(_sc_scatter_body)

    msg = pl.pallas_call(
        _msg_body,
        grid=(ehp // _EB,),
        in_specs=[_rows(_EB, _H), _rows(_EB, _H), _rows(_EB, de),
                  _full((de, de)), _full((1, de)),
                  _full((2 * _H, _H)), _full((de, _H)), _full((1, _H)),
                  _full((_H, _H)), _full((1, _H))],
        out_specs=_rows(_EB, _H),
        out_shape=jax.ShapeDtypeStruct((ehp, _H), f32),
    )

    upd_mid = pl.pallas_call(
        _upd_mid_body,
        grid=(nb,),
        in_specs=[_rows(_NB, _H)] * 5 + [
            _full((2 * _H, _H)), _full((1, _H)),
            _full((_H, _H)), _full((1, _H))],
        out_specs=_rows(_NB, _H),
        out_shape=jax.ShapeDtypeStruct((n, _H), f32),
    )

    hh = _H // 2
    upd_last = pl.pallas_call(
        _upd_last_body,
        grid=(nb,),
        in_specs=[_rows(_NB, _H)] * 5 + [
            _full((2 * _H, _H)), _full((1, _H)),
            _full((_H, _H)), _full((1, _H)),
            _full((_H, hh)), _full((1, hh)), _full((hh, 1)), _full((1, 1))],
        out_specs=[_rows(_NB, _H), _rows(_NB, 1)],
        out_shape=[jax.ShapeDtypeStruct((n, _H), f32),
                   jax.ShapeDtypeStruct((n, 1), f32)],
    )

    zero_n = jnp.zeros((n, _H), f32)
    npad = ehp - eh
    padi = jnp.zeros((npad,), jnp.int32)
    pads = n + jax.lax.rem(jax.lax.iota(jnp.int32, npad), _NPAD)
    pade = jnp.zeros((npad, de), f32)
    dstg = (jnp.concatenate([dst[:eh], padi]), jnp.concatenate([dst[eh:], padi]))
    dsts = (jnp.concatenate([dst[:eh], pads]), jnp.concatenate([dst[eh:], pads]))
    srcg = (jnp.concatenate([src[:eh], padi]), jnp.concatenate([src[eh:], padi]))
    eah = (jnp.concatenate([edge_attr[:eh], pade]),
           jnp.concatenate([edge_attr[eh:], pade]))

    for li, layer in enumerate(layers):
        w1, b1 = layer['msg1']
        w2, b2 = layer['msg2']
        g0 = sc_gather(h, dstg[0], srcg[0])
        g1 = sc_gather(h, dstg[1], srcg[1])
        m2_0 = msg(g0[0], g0[1], eah[0], ew, r2(eb), w1[:2 * _H],
                   w1[2 * _H:], r2(b1), w2, r2(b2))
        m2_1 = msg(g1[0], g1[1], eah[1], ew, r2(eb), w1[:2 * _H],
                   w1[2 * _H:], r2(b1), w2, r2(b2))
        ag0 = sc_scatter(m2_0, dsts[0], zero_n)
        ag1 = sc_scatter(m2_1, dsts[1], zero_n)
        uw1, ub1 = layer['up1']
        uw2, ub2 = layer['up2']
        if li < 2:
            h = upd_mid(h, ag0[0], ag0[1], ag1[0], ag1[1],
                        uw1, r2(ub1), uw2, r2(ub2))
        else:
            h, lg = upd_last(h, ag0[0], ag0[1], ag1[0], ag1[1],
                             uw1, r2(ub1), uw2, r2(ub2),
                             tw1, r2(tb1), tw2, r2(tb2))

    return lg.reshape(-1), h


def kernel(x, edge_index, edge_attr, params):
    src = edge_index[0]
    dst = edge_index[1]
    return _run(x, src, dst, edge_attr, params)
